# Initial kernel scaffold; baseline (speedup 1.0000x reference)
import jax
import jax.numpy as jnp
from jax.experimental import pallas as pl


def kernel(x, pos, batch, W_pre, b_pre, kernel, kernel_weight, W_post, b_post, W_sc, b_sc):
    raise NotImplementedError("write your pallas kernel here")


# speedup vs baseline: 2.4295x; 2.4295x over previous
import functools

import jax
import jax.numpy as jnp
from jax.experimental import pallas as pl
from jax.experimental.pallas import tpu as pltpu

_N = 10000
_IN_F = 128
_OUT_F = 128
_C = 64
_RATIO = 0.25
_RADIUS = 0.1
_MAX_NB = 64
_NUM_KP = 16
_KP_EXTENT = _RADIUS / 1.5
_M = 2500
_LANES = 128
_ROWS = 80
_NPAD = _ROWS * _LANES


def _fps_body(posx, posy, posz, d0, out_ref, dists):
    dists[...] = d0[...]
    row = jax.lax.broadcasted_iota(jnp.int32, (_ROWS, _LANES), 0)
    col = jax.lax.broadcasted_iota(jnp.int32, (_ROWS, _LANES), 1)
    flat = row * _LANES + col
    out_ref[0] = 0

    def body(i, carry):
        lx, ly, lz = carry
        dx = posx[...] - lx
        dy = posy[...] - ly
        dz = posz[...] - lz
        d = (dx * dx + dy * dy) + dz * dz
        nd = jnp.minimum(dists[...], d)
        dists[...] = nd
        m = jnp.max(nd)
        sel = nd == m
        j = jnp.min(jnp.where(sel, flat, _NPAD))
        out_ref[i] = j
        m2 = flat == j
        nlx = jnp.sum(jnp.where(m2, posx[...], 0.0))
        nly = jnp.sum(jnp.where(m2, posy[...], 0.0))
        nlz = jnp.sum(jnp.where(m2, posz[...], 0.0))
        return (nlx, nly, nlz)

    jax.lax.fori_loop(1, _M, body, (posx[0, 0], posy[0, 0], posz[0, 0]))


def _fps(posx, posy, posz, d0):
    return pl.pallas_call(
        _fps_body,
        out_shape=jax.ShapeDtypeStruct((_M,), jnp.int32),
        in_specs=[pl.BlockSpec(memory_space=pltpu.VMEM)] * 4,
        out_specs=pl.BlockSpec(memory_space=pltpu.SMEM),
        scratch_shapes=[pltpu.VMEM((_ROWS, _LANES), jnp.float32)],
    )(posx, posy, posz, d0)


def _fps_idx(pos):
    pad = _NPAD - _N
    px = jnp.pad(pos[:, 0], (0, pad)).reshape(_ROWS, _LANES)
    py = jnp.pad(pos[:, 1], (0, pad)).reshape(_ROWS, _LANES)
    pz = jnp.pad(pos[:, 2], (0, pad)).reshape(_ROWS, _LANES)
    valid = (jnp.arange(_NPAD) < _N).reshape(_ROWS, _LANES)
    d0 = jnp.where(valid, jnp.inf, -jnp.inf).astype(jnp.float32)
    return _fps(px, py, pz, d0)


def kernel(x, pos, batch, W_pre, b_pre, kernel, kernel_weight, W_post,
           b_post, W_sc, b_sc):
    idx = _fps_idx(pos)

    qpos = pos[idx]
    d2 = jnp.sum((qpos[:, None, :] - pos[None, :, :]) ** 2, axis=-1)
    negd, cols = jax.lax.top_k(-d2, _MAX_NB)
    mask = (-negd) <= _RADIUS * _RADIUS
    row = jnp.broadcast_to(
        jnp.arange(_M, dtype=jnp.int32)[:, None], cols.shape).reshape(-1)
    col = cols.reshape(-1).astype(jnp.int32)
    mask = mask.reshape(-1)

    x_side = x @ W_pre + b_pre
    pos_i = qpos[row]
    pos_j = pos[col]
    x_j = x_side[col]
    neighbors = pos_j - pos_i
    kp = kernel.reshape(-1, 3)
    differences = neighbors[:, None, :] - kp[None, :, :]
    sq_distances = jnp.sum(differences ** 2, axis=-1)
    all_weights = jnp.maximum(1.0 - sq_distances / (_KP_EXTENT ** 2), 0.0)
    nn1 = jnp.argmin(sq_distances, axis=-1)
    w = jnp.take_along_axis(all_weights, nn1[:, None], axis=1)
    Kw = jnp.take(kernel_weight, nn1, axis=0)
    weighted_features = w * x_j
    msg = jnp.einsum('na,nac->nc', weighted_features, Kw)
    msg = msg * mask[:, None].astype(msg.dtype)
    aggr = jax.ops.segment_sum(msg, row, num_segments=_M)
    x_side_out = aggr @ W_post + b_post
    x_shortcut = jnp.take(x, idx, axis=0) @ W_sc + b_sc
    out = x_side_out + x_shortcut
    return out, jnp.take(pos, idx, axis=0), jnp.take(batch, idx, axis=0)


# speedup vs baseline: 6.8804x; 2.8320x over previous
import functools

import jax
import jax.numpy as jnp
from jax import lax
from jax.experimental import pallas as pl
from jax.experimental.pallas import tpu as pltpu
from jax.experimental.pallas import tpu_sc as plsc

_N = 10000
_IN_F = 128
_OUT_F = 128
_C = 64
_RADIUS = 0.1
_MAX_NB = 64
_NUM_KP = 16
_KP_EXTENT = _RADIUS / 1.5
_M = 2500
_LANES = 128
_ROWS = 80
_NPAD = _ROWS * _LANES
_TPAD = _NPAD + 16

_NW = 32
_QPT = 80
_MPAD = _NW * _QPT
_CAP = 256
_CLAMP = 224
_NCHUNK = _NPAD // 16

_R2 = _RADIUS * _RADIUS
_EXT2 = _KP_EXTENT ** 2


def _fps_body(posx, posy, posz, d0, out_ref, dists):
    dists[...] = d0[...]
    row = jax.lax.broadcasted_iota(jnp.int32, (_ROWS, _LANES), 0)
    col = jax.lax.broadcasted_iota(jnp.int32, (_ROWS, _LANES), 1)
    flat = row * _LANES + col
    out_ref[0] = 0

    def body(i, carry):
        lx, ly, lz = carry
        dx = posx[...] - lx
        dy = posy[...] - ly
        dz = posz[...] - lz
        d = (dx * dx + dy * dy) + dz * dz
        nd = jnp.minimum(dists[...], d)
        dists[...] = nd
        m = jnp.max(nd)
        sel = nd == m
        j = jnp.min(jnp.where(sel, flat, _NPAD))
        out_ref[i] = j
        m2 = flat == j
        nlx = jnp.sum(jnp.where(m2, posx[...], 0.0))
        nly = jnp.sum(jnp.where(m2, posy[...], 0.0))
        nlz = jnp.sum(jnp.where(m2, posz[...], 0.0))
        return (nlx, nly, nlz)

    jax.lax.fori_loop(1, _M, body, (posx[0, 0], posy[0, 0], posz[0, 0]))


def _fps(posx, posy, posz, d0):
    return pl.pallas_call(
        _fps_body,
        out_shape=jax.ShapeDtypeStruct((_M,), jnp.int32),
        in_specs=[pl.BlockSpec(memory_space=pltpu.VMEM)] * 4,
        out_specs=pl.BlockSpec(memory_space=pltpu.SMEM),
        scratch_shapes=[pltpu.VMEM((_ROWS, _LANES), jnp.float32)],
    )(posx, posy, posz, d0)


def _fps_idx(pos):
    pad = _NPAD - _N
    px = jnp.pad(pos[:, 0], (0, pad)).reshape(_ROWS, _LANES)
    py = jnp.pad(pos[:, 1], (0, pad)).reshape(_ROWS, _LANES)
    pz = jnp.pad(pos[:, 2], (0, pad)).reshape(_ROWS, _LANES)
    valid = (jnp.arange(_NPAD) < _N).reshape(_ROWS, _LANES)
    d0 = jnp.where(valid, jnp.inf, -jnp.inf).astype(jnp.float32)
    return _fps(px, py, pz, d0)


def _pre_body(x_ref, w_ref, b_ref, o_ref):
    o_ref[...] = (jnp.dot(x_ref[...], w_ref[...],
                          preferred_element_type=jnp.float32) + b_ref[...])


def _pre_matmul(x, W_pre, b_pre):
    blk = 1000
    Wp = jnp.pad(W_pre, ((0, 0), (0, _LANES - _C)))
    bp = jnp.pad(b_pre, (0, _LANES - _C)).reshape(1, _LANES)
    return pl.pallas_call(
        _pre_body,
        grid=(_N // blk,),
        in_specs=[
            pl.BlockSpec((blk, _IN_F), lambda i: (i, 0)),
            pl.BlockSpec((_IN_F, _LANES), lambda i: (0, 0)),
            pl.BlockSpec((1, _LANES), lambda i: (0, 0)),
        ],
        out_specs=pl.BlockSpec((blk, _LANES), lambda i: (i, 0)),
        out_shape=jax.ShapeDtypeStruct((_N, _LANES), jnp.float32),
    )(x, Wp, bp)


def _sc_stage(posx, posy, posz, batch_pad, idx_pad, x_side, x, kpx, kpy, kpz):
    mesh = plsc.VectorSubcoreMesh(core_axis_name="c", subcore_axis_name="s")

    @functools.partial(
        pl.kernel,
        out_type=(
            jax.ShapeDtypeStruct((_MPAD, _NUM_KP * _C), jnp.float32),
            jax.ShapeDtypeStruct((_MPAD, _IN_F), jnp.float32),
            jax.ShapeDtypeStruct((_MPAD,), jnp.float32),
            jax.ShapeDtypeStruct((_MPAD,), jnp.float32),
            jax.ShapeDtypeStruct((_MPAD,), jnp.float32),
            jax.ShapeDtypeStruct((_MPAD,), jnp.int32),
        ),
        mesh=mesh,
        compiler_params=pltpu.CompilerParams(needs_layout_passes=False),
        scratch_types=[
            pltpu.VMEM((_TPAD,), jnp.float32),
            pltpu.VMEM((_TPAD,), jnp.float32),
            pltpu.VMEM((_TPAD,), jnp.float32),
            pltpu.VMEM((_NPAD,), jnp.int32),
            pltpu.VMEM((_QPT,), jnp.int32),
            pltpu.VMEM((_QPT + 16,), jnp.float32),
            pltpu.VMEM((_QPT + 16,), jnp.float32),
            pltpu.VMEM((_QPT + 16,), jnp.float32),
            pltpu.VMEM((_QPT,), jnp.int32),
            pltpu.VMEM((_QPT, _IN_F), jnp.float32),
            pltpu.VMEM((_CAP,), jnp.int32),
            pltpu.VMEM((_CAP,), jnp.float32),
            pltpu.VMEM((_QPT,), jnp.int32),
            pltpu.VMEM((_QPT, _LANES), jnp.float32),
            pltpu.VMEM((_NUM_KP * _C,), jnp.float32),
            pltpu.VMEM((16,), jnp.float32),
            pltpu.VMEM((16,), jnp.float32),
            pltpu.VMEM((16,), jnp.float32),
            pltpu.SemaphoreType.DMA,
        ],
    )
    def sc_kernel(posx_h, posy_h, posz_h, batch_h, idx_h, xside_h, x_h,
                  kpx_h, kpy_h, kpz_h,
                  S_h, xg_h, pox_h, poy_h, poz_h, pob_h,
                  posx_v, posy_v, posz_v, batch_v, idx_v,
                  qx_v, qy_v, qz_v, qb_v, xg_v,
                  cand_v, d2s_v, cand64_v, xrows_v, Sq_v,
                  kpx_v, kpy_v, kpz_v, dsem):
        wid = lax.axis_index("s") * 2 + lax.axis_index("c")
        base = wid * _QPT

        pltpu.sync_copy(posx_h, posx_v)
        pltpu.sync_copy(posy_h, posy_v)
        pltpu.sync_copy(posz_h, posz_v)
        pltpu.sync_copy(batch_h, batch_v)
        pltpu.sync_copy(idx_h.at[pl.ds(base, _QPT)], idx_v)
        pltpu.sync_copy(kpx_h, kpx_v)
        pltpu.sync_copy(kpy_h, kpy_v)
        pltpu.sync_copy(kpz_h, kpz_v)

        iota16 = lax.iota(jnp.int32, 16)
        zero16f = jnp.zeros((16,), jnp.float32)
        zero16i = jnp.zeros((16,), jnp.int32)
        r2 = jnp.float32(_R2)
        rext2 = jnp.float32(1.0 / _EXT2)

        for g in range(_QPT // 16):
            iv = idx_v[pl.ds(g * 16, 16)]
            qx_v[pl.ds(g * 16, 16)] = plsc.load_gather(posx_v, [iv])
            qy_v[pl.ds(g * 16, 16)] = plsc.load_gather(posy_v, [iv])
            qz_v[pl.ds(g * 16, 16)] = plsc.load_gather(posz_v, [iv])
            qb_v[pl.ds(g * 16, 16)] = plsc.load_gather(batch_v, [iv])
        pltpu.sync_copy(qx_v.at[pl.ds(0, _QPT)], pox_h.at[pl.ds(base, _QPT)])
        pltpu.sync_copy(qy_v.at[pl.ds(0, _QPT)], poy_h.at[pl.ds(base, _QPT)])
        pltpu.sync_copy(qz_v.at[pl.ds(0, _QPT)], poz_h.at[pl.ds(base, _QPT)])
        pltpu.sync_copy(qb_v, pob_h.at[pl.ds(base, _QPT)])

        pltpu.async_copy(x_h.at[idx_v], xg_v, dsem).wait()
        pltpu.sync_copy(xg_v, xg_h.at[pl.ds(base, _QPT)])

        kpx16 = kpx_v[...]
        kpy16 = kpy_v[...]
        kpz16 = kpz_v[...]

        def per_query(ql, _):
            qg = base + ql
            qx = qx_v[pl.ds(ql, 16)][0]
            qy = qy_v[pl.ds(ql, 16)][0]
            qz = qz_v[pl.ds(ql, 16)][0]

            for t in range(_MAX_NB // 16):
                cand_v[pl.ds(t * 16, 16)] = zero16i
            for t in range(_QPT // 16):
                cand64_v[pl.ds(t * 16, 16)] = zero16i

            def chunk(c, cnt):
                of = c * 16
                px = posx_v[pl.ds(of, 16)]
                py = posy_v[pl.ds(of, 16)]
                pz = posz_v[pl.ds(of, 16)]
                dx = px - qx
                dy = py - qy
                dz = pz - qz
                d2 = (dx * dx + dy * dy) + dz * dz
                msk = d2 <= r2
                plsc.store_compressed(cand_v.at[pl.ds(cnt, 16)],
                                      iota16 + of, mask=msk)
                plsc.store_compressed(d2s_v.at[pl.ds(cnt, 16)], d2, mask=msk)
                pc = jnp.sum(msk.astype(jnp.int32))
                return jnp.minimum(cnt + pc, _CLAMP)

            cnt = lax.fori_loop(0, _NCHUNK, chunk, jnp.int32(0))

            @pl.when(cnt > _MAX_NB)
            def _rare():
                nch = (_CLAMP + 15) // 16

                def extract(k, _):
                    def mn(b, m):
                        db = d2s_v[pl.ds(b * 16, 16)]
                        valid = (iota16 + b * 16) < cnt
                        return jnp.minimum(
                            m, jnp.min(jnp.where(valid, db, jnp.inf)))
                    m = lax.fori_loop(0, nch, mn, jnp.float32(jnp.inf))

                    def fpos(b, p):
                        db = d2s_v[pl.ds(b * 16, 16)]
                        valid = (iota16 + b * 16) < cnt
                        cp = jnp.min(jnp.where((db == m) & valid,
                                               iota16 + b * 16, jnp.int32(10 ** 6)))
                        return jnp.minimum(p, cp)
                    p = lax.fori_loop(0, nch, fpos, jnp.int32(10 ** 6))
                    val = cand_v[pl.ds(p, 16)][0]
                    lane0 = iota16 == 0
                    plsc.store_scatter(cand64_v, [zero16i + k],
                                       zero16i + val, mask=lane0)
                    plsc.store_scatter(d2s_v, [zero16i + p],
                                       zero16f + jnp.inf, mask=lane0)
                    return 0

                lax.fori_loop(0, _MAX_NB, extract, 0)

            @pl.when(cnt <= _MAX_NB)
            def _common():
                for t in range(_MAX_NB // 16):
                    cand64_v[pl.ds(t * 16, 16)] = cand_v[pl.ds(t * 16, 16)]

            cnt64 = jnp.minimum(cnt, _MAX_NB)

            pltpu.async_copy(xside_h.at[cand64_v], xrows_v, dsem).wait()

            def zloop(t, _):
                Sq_v[pl.ds(t * 16, 16)] = zero16f
                return 0
            lax.fori_loop(0, _NUM_KP * _C // 16, zloop, 0)

            def slot(s, _):
                j = cand64_v[pl.ds(s, 16)][0]
                valid = s < cnt64
                dxe = posx_v[pl.ds(j, 16)][0] - qx
                dye = posy_v[pl.ds(j, 16)][0] - qy
                dze = posz_v[pl.ds(j, 16)][0] - qz
                dvx = dxe - kpx16
                dvy = dye - kpy16
                dvz = dze - kpz16
                sq = (dvx * dvx + dvy * dvy) + dvz * dvz
                minv = jnp.min(sq)
                nn = jnp.min(jnp.where(sq == minv, iota16, jnp.int32(16)))
                w = jnp.maximum(1.0 - minv * rext2, 0.0)
                w = jnp.where(valid, w, jnp.float32(0.0))
                off = nn * _C
                for t in range(_C // 16):
                    xv = xrows_v[s, pl.ds(t * 16, 16)]
                    plsc.addupdate(Sq_v.at[pl.ds(off + t * 16, 16)], w * xv)
                return 0

            lax.fori_loop(0, _MAX_NB, slot, 0)
            pltpu.sync_copy(Sq_v, S_h.at[qg])
            return 0

        lax.fori_loop(0, _QPT, per_query, 0)

    return sc_kernel(posx, posy, posz, batch_pad, idx_pad, x_side, x,
                     kpx, kpy, kpz)


def _tail_body(S_ref, kw_ref, wp_ref, xg_ref, wsc_ref, b_ref, o_ref):
    aggr = jnp.dot(S_ref[...], kw_ref[...], preferred_element_type=jnp.float32)
    o = jnp.dot(aggr, wp_ref[...], preferred_element_type=jnp.float32)
    o = o + jnp.dot(xg_ref[...], wsc_ref[...],
                    preferred_element_type=jnp.float32)
    o_ref[...] = o + b_ref[...]


def _tail(S, KWf, W_post, xg, W_sc, bsum):
    blk = 512
    return pl.pallas_call(
        _tail_body,
        grid=(_MPAD // blk,),
        in_specs=[
            pl.BlockSpec((blk, _NUM_KP * _C), lambda i: (i, 0)),
            pl.BlockSpec((_NUM_KP * _C, _C), lambda i: (0, 0)),
            pl.BlockSpec((_C, _OUT_F), lambda i: (0, 0)),
            pl.BlockSpec((blk, _IN_F), lambda i: (i, 0)),
            pl.BlockSpec((_IN_F, _OUT_F), lambda i: (0, 0)),
            pl.BlockSpec((1, _OUT_F), lambda i: (0, 0)),
        ],
        out_specs=pl.BlockSpec((blk, _OUT_F), lambda i: (i, 0)),
        out_shape=jax.ShapeDtypeStruct((_MPAD, _OUT_F), jnp.float32),
    )(S, KWf, W_post, xg, W_sc, bsum)


def kernel(x, pos, batch, W_pre, b_pre, kernel, kernel_weight, W_post,
           b_post, W_sc, b_sc):
    idx = _fps_idx(pos)
    idx_pad = jnp.pad(idx, (0, _MPAD - _M))

    x_side = _pre_matmul(x, W_pre, b_pre)

    big = jnp.full((_TPAD - _N,), 1e9, dtype=jnp.float32)
    posx = jnp.concatenate([pos[:, 0], big])
    posy = jnp.concatenate([pos[:, 1], big])
    posz = jnp.concatenate([pos[:, 2], big])
    batch_pad = jnp.pad(batch, (0, _NPAD - _N))
    kpx = kernel[0, :, 0]
    kpy = kernel[0, :, 1]
    kpz = kernel[0, :, 2]

    S, xg, pox, poy, poz, pob = _sc_stage(
        posx, posy, posz, batch_pad, idx_pad, x_side, x, kpx, kpy, kpz)

    KWf = kernel_weight.reshape(_NUM_KP * _C, _C)
    bsum = (b_post + b_sc).reshape(1, _OUT_F)
    out = _tail(S, KWf, W_post, xg, W_sc, bsum)[:_M]

    pos_out = jnp.stack([pox[:_M], poy[:_M], poz[:_M]], axis=1)
    batch_out = pob[:_M]
    return out, pos_out, batch_out


# speedup vs baseline: 6.9182x; 1.0055x over previous
import functools

import jax
import jax.numpy as jnp
from jax import lax
from jax.experimental import pallas as pl
from jax.experimental.pallas import tpu as pltpu
from jax.experimental.pallas import tpu_sc as plsc

_N = 10000
_IN_F = 128
_OUT_F = 128
_C = 64
_RADIUS = 0.1
_MAX_NB = 64
_NUM_KP = 16
_KP_EXTENT = _RADIUS / 1.5
_M = 2500
_LANES = 128
_ROWS = 80
_NPAD = _ROWS * _LANES
_TPAD = _NPAD + 16

_NW = 32
_QPT = 80
_MPAD = _NW * _QPT
_CAP = 256
_CLAMP = 224
_NCHUNK = _NPAD // 16

_R2 = _RADIUS * _RADIUS
_EXT2 = _KP_EXTENT ** 2


def _fps_body(posx, posy, posz, d0, out_ref, dists):
    dists[...] = d0[...]
    row = jax.lax.broadcasted_iota(jnp.int32, (_ROWS, _LANES), 0)
    col = jax.lax.broadcasted_iota(jnp.int32, (_ROWS, _LANES), 1)
    flat = row * _LANES + col
    out_ref[0] = 0

    def body(i, carry):
        lx, ly, lz = carry
        dx = posx[...] - lx
        dy = posy[...] - ly
        dz = posz[...] - lz
        d = (dx * dx + dy * dy) + dz * dz
        nd = jnp.minimum(dists[...], d)
        dists[...] = nd
        m = jnp.max(nd)
        sel = nd == m
        j = jnp.min(jnp.where(sel, flat, _NPAD))
        out_ref[i] = j
        m2 = flat == j
        nlx = jnp.sum(jnp.where(m2, posx[...], 0.0))
        nly = jnp.sum(jnp.where(m2, posy[...], 0.0))
        nlz = jnp.sum(jnp.where(m2, posz[...], 0.0))
        return (nlx, nly, nlz)

    jax.lax.fori_loop(1, _M, body, (posx[0, 0], posy[0, 0], posz[0, 0]))


def _fps(posx, posy, posz, d0):
    return pl.pallas_call(
        _fps_body,
        out_shape=jax.ShapeDtypeStruct((_M,), jnp.int32),
        in_specs=[pl.BlockSpec(memory_space=pltpu.VMEM)] * 4,
        out_specs=pl.BlockSpec(memory_space=pltpu.SMEM),
        scratch_shapes=[pltpu.VMEM((_ROWS, _LANES), jnp.float32)],
    )(posx, posy, posz, d0)


def _fps_idx(pos):
    pad = _NPAD - _N
    px = jnp.pad(pos[:, 0], (0, pad)).reshape(_ROWS, _LANES)
    py = jnp.pad(pos[:, 1], (0, pad)).reshape(_ROWS, _LANES)
    pz = jnp.pad(pos[:, 2], (0, pad)).reshape(_ROWS, _LANES)
    valid = (jnp.arange(_NPAD) < _N).reshape(_ROWS, _LANES)
    d0 = jnp.where(valid, jnp.inf, -jnp.inf).astype(jnp.float32)
    return _fps(px, py, pz, d0)


def _pre_body(x_ref, w_ref, b_ref, o_ref):
    o_ref[...] = (jnp.dot(x_ref[...], w_ref[...],
                          preferred_element_type=jnp.float32) + b_ref[...])


def _pre_matmul(x, W_pre, b_pre):
    blk = 1000
    Wp = jnp.pad(W_pre, ((0, 0), (0, _LANES - _C)))
    bp = jnp.pad(b_pre, (0, _LANES - _C)).reshape(1, _LANES)
    return pl.pallas_call(
        _pre_body,
        grid=(_N // blk,),
        in_specs=[
            pl.BlockSpec((blk, _IN_F), lambda i: (i, 0)),
            pl.BlockSpec((_IN_F, _LANES), lambda i: (0, 0)),
            pl.BlockSpec((1, _LANES), lambda i: (0, 0)),
        ],
        out_specs=pl.BlockSpec((blk, _LANES), lambda i: (i, 0)),
        out_shape=jax.ShapeDtypeStruct((_N, _LANES), jnp.float32),
    )(x, Wp, bp)


def _sc_stage(posx, posy, posz, batch_pad, idx_pad, x_side, x, kpx, kpy, kpz):
    mesh = plsc.VectorSubcoreMesh(core_axis_name="c", subcore_axis_name="s")

    @functools.partial(
        pl.kernel,
        out_type=(
            jax.ShapeDtypeStruct((_MPAD, _NUM_KP * _C), jnp.float32),
            jax.ShapeDtypeStruct((_MPAD, _IN_F), jnp.float32),
            jax.ShapeDtypeStruct((_MPAD,), jnp.float32),
            jax.ShapeDtypeStruct((_MPAD,), jnp.float32),
            jax.ShapeDtypeStruct((_MPAD,), jnp.float32),
            jax.ShapeDtypeStruct((_MPAD,), jnp.int32),
        ),
        mesh=mesh,
        compiler_params=pltpu.CompilerParams(needs_layout_passes=False),
        scratch_types=[
            pltpu.VMEM((_TPAD,), jnp.float32),
            pltpu.VMEM((_TPAD,), jnp.float32),
            pltpu.VMEM((_TPAD,), jnp.float32),
            pltpu.VMEM((_NPAD,), jnp.int32),
            pltpu.VMEM((_QPT,), jnp.int32),
            pltpu.VMEM((_QPT + 16,), jnp.float32),
            pltpu.VMEM((_QPT + 16,), jnp.float32),
            pltpu.VMEM((_QPT + 16,), jnp.float32),
            pltpu.VMEM((_QPT,), jnp.int32),
            pltpu.VMEM((_QPT, _IN_F), jnp.float32),
            pltpu.VMEM((_NCHUNK,), jnp.int32),
            pltpu.VMEM((_CAP,), jnp.int32),
            pltpu.VMEM((_CAP,), jnp.int32),
            pltpu.VMEM((_CAP,), jnp.float32),
            pltpu.VMEM((_QPT,), jnp.int32),
            pltpu.VMEM((_QPT, _LANES), jnp.float32),
            pltpu.VMEM((_NUM_KP * _C,), jnp.float32),
            pltpu.VMEM((16,), jnp.float32),
            pltpu.VMEM((16,), jnp.float32),
            pltpu.VMEM((16,), jnp.float32),
            pltpu.SemaphoreType.DMA,
        ],
    )
    def sc_kernel(posx_h, posy_h, posz_h, batch_h, idx_h, xside_h, x_h,
                  kpx_h, kpy_h, kpz_h,
                  S_h, xg_h, pox_h, poy_h, poz_h, pob_h,
                  posx_v, posy_v, posz_v, batch_v, idx_v,
                  qx_v, qy_v, qz_v, qb_v, xg_v,
                  actflg_v, act_v, cand_v, d2s_v, cand64_v, xrows_v, Sq_v,
                  kpx_v, kpy_v, kpz_v, dsem):
        wid = lax.axis_index("s") * 2 + lax.axis_index("c")
        base = wid * _QPT

        pltpu.sync_copy(posx_h, posx_v)
        pltpu.sync_copy(posy_h, posy_v)
        pltpu.sync_copy(posz_h, posz_v)
        pltpu.sync_copy(batch_h, batch_v)
        pltpu.sync_copy(idx_h.at[pl.ds(base, _QPT)], idx_v)
        pltpu.sync_copy(kpx_h, kpx_v)
        pltpu.sync_copy(kpy_h, kpy_v)
        pltpu.sync_copy(kpz_h, kpz_v)

        iota16 = lax.iota(jnp.int32, 16)
        zero16f = jnp.zeros((16,), jnp.float32)
        zero16i = jnp.zeros((16,), jnp.int32)
        r2 = jnp.float32(_R2)
        rext2 = jnp.float32(1.0 / _EXT2)

        for g in range(_QPT // 16):
            iv = idx_v[pl.ds(g * 16, 16)]
            qx_v[pl.ds(g * 16, 16)] = plsc.load_gather(posx_v, [iv])
            qy_v[pl.ds(g * 16, 16)] = plsc.load_gather(posy_v, [iv])
            qz_v[pl.ds(g * 16, 16)] = plsc.load_gather(posz_v, [iv])
            qb_v[pl.ds(g * 16, 16)] = plsc.load_gather(batch_v, [iv])
        pltpu.sync_copy(qx_v.at[pl.ds(0, _QPT)], pox_h.at[pl.ds(base, _QPT)])
        pltpu.sync_copy(qy_v.at[pl.ds(0, _QPT)], poy_h.at[pl.ds(base, _QPT)])
        pltpu.sync_copy(qz_v.at[pl.ds(0, _QPT)], poz_h.at[pl.ds(base, _QPT)])
        pltpu.sync_copy(qb_v, pob_h.at[pl.ds(base, _QPT)])

        pltpu.async_copy(x_h.at[idx_v], xg_v, dsem).wait()
        pltpu.sync_copy(xg_v, xg_h.at[pl.ds(base, _QPT)])

        kpx16 = kpx_v[...]
        kpy16 = kpy_v[...]
        kpz16 = kpz_v[...]

        def per_query(ql, _):
            qg = base + ql
            qx = qx_v[pl.ds(ql, 16)][0]
            qy = qy_v[pl.ds(ql, 16)][0]
            qz = qz_v[pl.ds(ql, 16)][0]

            for t in range(_MAX_NB // 16):
                cand_v[pl.ds(t * 16, 16)] = zero16i
            for t in range(_QPT // 16):
                cand64_v[pl.ds(t * 16, 16)] = zero16i

            @plsc.parallel_loop(0, _NCHUNK // 16, step=1)
            def _p1(g):
                flg = zero16i
                for k in range(16):
                    of = g * 256 + k * 16
                    px = posx_v[pl.ds(of, 16)]
                    py = posy_v[pl.ds(of, 16)]
                    pz = posz_v[pl.ds(of, 16)]
                    dx = px - qx
                    dy = py - qy
                    dz = pz - qz
                    d2 = (dx * dx + dy * dy) + dz * dz
                    msk = d2 <= r2
                    pc = plsc.all_reduce_population_count(msk)
                    flg = flg + jnp.where(iota16 == k, pc, 0)
                actflg_v[pl.ds(g * 16, 16)] = flg

            def p2a(v, na):
                fv = actflg_v[pl.ds(v * 16, 16)]
                am = fv > 0
                plsc.store_compressed(act_v.at[pl.ds(na, 16)],
                                      iota16 + v * 16, mask=am)
                return jnp.minimum(na + jnp.sum(am.astype(jnp.int32)), _CLAMP)

            na = lax.fori_loop(0, _NCHUNK // 16, p2a, jnp.int32(0))

            def p2b(a, cnt):
                c = act_v[pl.ds(a, 16)][0]
                of = c * 16
                px = posx_v[pl.ds(of, 16)]
                py = posy_v[pl.ds(of, 16)]
                pz = posz_v[pl.ds(of, 16)]
                dx = px - qx
                dy = py - qy
                dz = pz - qz
                d2 = (dx * dx + dy * dy) + dz * dz
                msk = d2 <= r2
                plsc.store_compressed(cand_v.at[pl.ds(cnt, 16)],
                                      iota16 + of, mask=msk)
                plsc.store_compressed(d2s_v.at[pl.ds(cnt, 16)], d2, mask=msk)
                pc = jnp.sum(msk.astype(jnp.int32))
                return jnp.minimum(cnt + pc, _CLAMP)

            cnt = lax.fori_loop(0, na, p2b, jnp.int32(0))

            @pl.when(cnt > _MAX_NB)
            def _rare():
                nch = (_CLAMP + 15) // 16

                def extract(k, _):
                    def mn(b, m):
                        db = d2s_v[pl.ds(b * 16, 16)]
                        valid = (iota16 + b * 16) < cnt
                        return jnp.minimum(
                            m, jnp.min(jnp.where(valid, db, jnp.inf)))
                    m = lax.fori_loop(0, nch, mn, jnp.float32(jnp.inf))

                    def fpos(b, p):
                        db = d2s_v[pl.ds(b * 16, 16)]
                        valid = (iota16 + b * 16) < cnt
                        cp = jnp.min(jnp.where((db == m) & valid,
                                               iota16 + b * 16, jnp.int32(10 ** 6)))
                        return jnp.minimum(p, cp)
                    p = lax.fori_loop(0, nch, fpos, jnp.int32(10 ** 6))
                    val = cand_v[pl.ds(p, 16)][0]
                    lane0 = iota16 == 0
                    plsc.store_scatter(cand64_v, [zero16i + k],
                                       zero16i + val, mask=lane0)
                    plsc.store_scatter(d2s_v, [zero16i + p],
                                       zero16f + jnp.inf, mask=lane0)
                    return 0

                lax.fori_loop(0, _MAX_NB, extract, 0)

            @pl.when(cnt <= _MAX_NB)
            def _common():
                for t in range(_MAX_NB // 16):
                    cand64_v[pl.ds(t * 16, 16)] = cand_v[pl.ds(t * 16, 16)]

            cnt64 = jnp.minimum(cnt, _MAX_NB)

            pltpu.async_copy(xside_h.at[cand64_v], xrows_v, dsem).wait()

            def zloop(t, _):
                Sq_v[pl.ds(t * 16, 16)] = zero16f
                return 0
            lax.fori_loop(0, _NUM_KP * _C // 16, zloop, 0)

            def slot(s, _):
                j = cand64_v[pl.ds(s, 16)][0]
                valid = s < cnt64
                dxe = posx_v[pl.ds(j, 16)][0] - qx
                dye = posy_v[pl.ds(j, 16)][0] - qy
                dze = posz_v[pl.ds(j, 16)][0] - qz
                dvx = dxe - kpx16
                dvy = dye - kpy16
                dvz = dze - kpz16
                sq = (dvx * dvx + dvy * dvy) + dvz * dvz
                minv = jnp.min(sq)
                nn = jnp.min(jnp.where(sq == minv, iota16, jnp.int32(16)))
                w = jnp.maximum(1.0 - minv * rext2, 0.0)
                w = jnp.where(valid, w, jnp.float32(0.0))
                off = nn * _C
                for t in range(_C // 16):
                    xv = xrows_v[s, pl.ds(t * 16, 16)]
                    plsc.addupdate(Sq_v.at[pl.ds(off + t * 16, 16)], w * xv)
                return 0

            lax.fori_loop(0, _MAX_NB, slot, 0)
            pltpu.sync_copy(Sq_v, S_h.at[qg])
            return 0

        lax.fori_loop(0, _QPT, per_query, 0)

    return sc_kernel(posx, posy, posz, batch_pad, idx_pad, x_side, x,
                     kpx, kpy, kpz)


def _tail_body(S_ref, kw_ref, wp_ref, xg_ref, wsc_ref, b_ref, o_ref):
    aggr = jnp.dot(S_ref[...], kw_ref[...], preferred_element_type=jnp.float32)
    o = jnp.dot(aggr, wp_ref[...], preferred_element_type=jnp.float32)
    o = o + jnp.dot(xg_ref[...], wsc_ref[...],
                    preferred_element_type=jnp.float32)
    o_ref[...] = o + b_ref[...]


def _tail(S, KWf, W_post, xg, W_sc, bsum):
    blk = 512
    return pl.pallas_call(
        _tail_body,
        grid=(_MPAD // blk,),
        in_specs=[
            pl.BlockSpec((blk, _NUM_KP * _C), lambda i: (i, 0)),
            pl.BlockSpec((_NUM_KP * _C, _C), lambda i: (0, 0)),
            pl.BlockSpec((_C, _OUT_F), lambda i: (0, 0)),
            pl.BlockSpec((blk, _IN_F), lambda i: (i, 0)),
            pl.BlockSpec((_IN_F, _OUT_F), lambda i: (0, 0)),
            pl.BlockSpec((1, _OUT_F), lambda i: (0, 0)),
        ],
        out_specs=pl.BlockSpec((blk, _OUT_F), lambda i: (i, 0)),
        out_shape=jax.ShapeDtypeStruct((_MPAD, _OUT_F), jnp.float32),
    )(S, KWf, W_post, xg, W_sc, bsum)


def kernel(x, pos, batch, W_pre, b_pre, kernel, kernel_weight, W_post,
           b_post, W_sc, b_sc):
    idx = _fps_idx(pos)
    idx_pad = jnp.pad(idx, (0, _MPAD - _M))

    x_side = _pre_matmul(x, W_pre, b_pre)

    big = jnp.full((_TPAD - _N,), 1e9, dtype=jnp.float32)
    posx = jnp.concatenate([pos[:, 0], big])
    posy = jnp.concatenate([pos[:, 1], big])
    posz = jnp.concatenate([pos[:, 2], big])
    batch_pad = jnp.pad(batch, (0, _NPAD - _N))
    kpx = kernel[0, :, 0]
    kpy = kernel[0, :, 1]
    kpz = kernel[0, :, 2]

    S, xg, pox, poy, poz, pob = _sc_stage(
        posx, posy, posz, batch_pad, idx_pad, x_side, x, kpx, kpy, kpz)

    KWf = kernel_weight.reshape(_NUM_KP * _C, _C)
    bsum = (b_post + b_sc).reshape(1, _OUT_F)
    out = _tail(S, KWf, W_post, xg, W_sc, bsum)[:_M]

    pos_out = jnp.stack([pox[:_M], poy[:_M], poz[:_M]], axis=1)
    batch_out = pob[:_M]
    return out, pos_out, batch_out
